# Initial kernel scaffold; baseline (speedup 1.0000x reference)
import jax
import jax.numpy as jnp
from jax.experimental import pallas as pl


def kernel(start, mid, end, durations, max_frames):
    raise NotImplementedError("write your pallas kernel here")


# speedup vs baseline: 88.2082x; 88.2082x over previous
import functools

import jax
import jax.numpy as jnp
from jax import lax
from jax.experimental import pallas as pl
from jax.experimental.pallas import tpu as pltpu
from jax.experimental.pallas import tpu_sc as plsc

B, N, F = 8, 2048, 256
T = 8192
NW = 32
TPB = NW // B
CH = 64
NCH = T // (TPB * CH)
FPT = T // TPB

_mesh = plsc.VectorSubcoreMesh(core_axis_name="c", subcore_axis_name="s")


@functools.partial(
    pl.kernel,
    out_type=[
        jax.ShapeDtypeStruct((B * T, F), jnp.float32),
        jax.ShapeDtypeStruct((B * T,), jnp.int32),
    ],
    mesh=_mesh,
    compiler_params=pltpu.CompilerParams(needs_layout_passes=False),
    scratch_types=[
        pltpu.VMEM((N,), jnp.int32),
        pltpu.VMEM((N,), jnp.int32),
        pltpu.VMEM((FPT,), jnp.int32),
        pltpu.VMEM((FPT,), jnp.float32),
        pltpu.VMEM((FPT,), jnp.float32),
        pltpu.VMEM((FPT,), jnp.float32),
        pltpu.VMEM((FPT,), jnp.int32),
        pltpu.VMEM((16,), jnp.int32),
        pltpu.VMEM((CH, F), jnp.float32),
        pltpu.VMEM((CH, F), jnp.float32),
        pltpu.VMEM((CH, F), jnp.float32),
        pltpu.VMEM((CH, F), jnp.float32),
        pltpu.SemaphoreType.DMA,
    ],
)
def _sc_interp(st, mi, en, dur, mf, frames_o, mask_o,
               dur_v, csum_v, idxg_v, a_v, b_v, c_v, msk_v, mf_v,
               sbuf, mbuf, ebuf, zbuf, sem):
    cid = lax.axis_index("c")
    sid = lax.axis_index("s")
    wid = cid * 16 + sid
    b = wid // TPB
    sub = wid % TPB

    pltpu.sync_copy(dur.at[b], dur_v)
    pltpu.sync_copy(mf, mf_v)

    def cs_step(i, carry):
        v = dur_v[pl.ds(i * 16, 16)]
        csum_v[pl.ds(i * 16, 16)] = jnp.cumsum(v) + carry
        return carry + jnp.sum(v)

    total = lax.fori_loop(0, N // 16, cs_step, jnp.int32(0))
    vt_vec = jnp.minimum(jnp.full((16,), total, jnp.int32), mf_v[...])
    vt_s = jnp.max(vt_vec)

    def z_step(i, _):
        for k in range(F // 16):
            zbuf[i, pl.ds(k * 16, 16)] = jnp.zeros((16,), jnp.float32)
        return 0

    lax.fori_loop(0, CH, z_step, 0)

    def pre_step(i, _):
        l = i * 16
        j = l // CH
        w = l - j * CH
        g = sub + TPB * j
        tvec = g * CH + w + lax.iota(jnp.int32, 16)
        lo = jnp.zeros((16,), jnp.int32)
        hi = jnp.full((16,), N, jnp.int32)
        for _step in range(12):
            m = jnp.minimum((lo + hi) >> 1, N - 1)
            vals = plsc.load_gather(csum_v, [m])
            pred = vals <= tvec
            lo = jnp.where(pred, m + 1, lo)
            hi = jnp.where(pred, hi, m)
        idx_c = jnp.minimum(lo, N - 1)
        d = plsc.load_gather(dur_v, [idx_c])
        cs = plsc.load_gather(csum_v, [idx_c])
        p = tvec - (cs - d)
        half = d >> 1
        rem = d - half
        pf = p.astype(jnp.float32)
        halff = half.astype(jnp.float32)
        den1 = jnp.maximum(half - 1, 1).astype(jnp.float32)
        den2 = jnp.maximum(rem - 1, 1).astype(jnp.float32)
        t1 = jnp.where(half > 1, pf / den1, 0.0)
        t2 = jnp.where(rem > 1, (pf - halff) / den2, 0.0)
        s1 = p < half
        a3 = jnp.where(s1, 1.0 - t1, 0.0)
        b3 = jnp.where(s1, t1, 1.0 - t2)
        c3 = jnp.where(s1, 0.0, t2)
        a2 = jnp.where(p == 0, 1.0, 0.0)
        is1 = d == 1
        is2 = d == 2
        a = jnp.where(is1, 0.0, jnp.where(is2, a2, a3))
        bb = jnp.where(is1, 1.0, jnp.where(is2, 0.0, b3))
        c = jnp.where(is1, 0.0, jnp.where(is2, 1.0 - a2, c3))
        valid = tvec < vt_vec
        vf = jnp.where(valid, 1.0, 0.0)
        sl = pl.ds(l, 16)
        a_v[sl] = a * vf
        b_v[sl] = bb * vf
        c_v[sl] = c * vf
        msk_v[sl] = jnp.where(valid, 1, 0)
        idxg_v[sl] = b * N + idx_c
        return 0

    lax.fori_loop(0, FPT // 16, pre_step, 0)

    def chunk_step(j, _):
        g = sub + TPB * j
        t0 = g * CH
        row0 = b * T + t0
        loc = j * CH

        @pl.when(t0 < vt_s)
        def _valid_chunk():
            idx_sl = idxg_v.at[pl.ds(loc, CH)]
            cp1 = pltpu.async_copy(st.at[idx_sl], sbuf, sem)
            cp2 = pltpu.async_copy(mi.at[idx_sl], mbuf, sem)
            cp3 = pltpu.async_copy(en.at[idx_sl], ebuf, sem)
            cp1.wait()
            cp2.wait()
            cp3.wait()

            def f_step(fr, _c):
                iv = jnp.full((16,), loc + fr, jnp.int32)
                asp = plsc.load_gather(a_v, [iv])
                bsp = plsc.load_gather(b_v, [iv])
                csp = plsc.load_gather(c_v, [iv])
                for k in range(F // 16):
                    fsl = pl.ds(k * 16, 16)
                    blended = (sbuf[fr, fsl] * asp + mbuf[fr, fsl] * bsp
                               + ebuf[fr, fsl] * csp)
                    sbuf[fr, fsl] = blended
                return 0

            lax.fori_loop(0, CH, f_step, 0)
            pltpu.sync_copy(sbuf, frames_o.at[pl.ds(row0, CH)])

        @pl.when(t0 >= vt_s)
        def _zero_chunk():
            pltpu.sync_copy(zbuf, frames_o.at[pl.ds(row0, CH)])

        pltpu.sync_copy(msk_v.at[pl.ds(loc, CH)], mask_o.at[pl.ds(row0, CH)])
        return 0

    lax.fori_loop(0, NCH, chunk_step, 0)


def kernel(start, mid, end, durations, max_frames):
    st = start.reshape(B * N, F)
    mi = mid.reshape(B * N, F)
    en = end.reshape(B * N, F)
    dur = durations.astype(jnp.int32)
    mf = jnp.full((16,), jnp.asarray(max_frames, jnp.int32))
    frames_flat, mask_i = _sc_interp(st, mi, en, dur, mf)
    frames = frames_flat.reshape(B, T, F)
    mask = mask_i.reshape(B, T) != 0
    return frames, mask


# speedup vs baseline: 125.7575x; 1.4257x over previous
import functools

import jax
import jax.numpy as jnp
from jax import lax
from jax.experimental import pallas as pl
from jax.experimental.pallas import tpu as pltpu
from jax.experimental.pallas import tpu_sc as plsc

B, N, F = 8, 2048, 256
T = 8192
NW = 32
TPB = NW // B
CH = 64
NCH = T // (TPB * CH)
FPT = T // TPB

_mesh = plsc.VectorSubcoreMesh(core_axis_name="c", subcore_axis_name="s")


@functools.partial(
    pl.kernel,
    out_type=[
        jax.ShapeDtypeStruct((B * T, F), jnp.float32),
        jax.ShapeDtypeStruct((B * T,), jnp.int32),
    ],
    mesh=_mesh,
    compiler_params=pltpu.CompilerParams(needs_layout_passes=False),
    scratch_types=[
        pltpu.VMEM((N,), jnp.int32),
        pltpu.VMEM((N,), jnp.int32),
        pltpu.VMEM((FPT,), jnp.int32),
        pltpu.VMEM((FPT,), jnp.float32),
        pltpu.VMEM((FPT,), jnp.float32),
        pltpu.VMEM((FPT,), jnp.float32),
        pltpu.VMEM((FPT,), jnp.int32),
        pltpu.VMEM((16,), jnp.int32),
        pltpu.VMEM((CH, F), jnp.float32),
        pltpu.VMEM((CH, F), jnp.float32),
        pltpu.VMEM((CH, F), jnp.float32),
        pltpu.VMEM((CH, F), jnp.float32),
        pltpu.VMEM((CH, F), jnp.float32),
        pltpu.VMEM((CH, F), jnp.float32),
        pltpu.VMEM((CH, F), jnp.float32),
        pltpu.SemaphoreType.DMA,
        pltpu.SemaphoreType.DMA,
        pltpu.SemaphoreType.DMA,
        pltpu.SemaphoreType.DMA,
    ],
)
def _sc_interp(st, mi, en, dur, mf, frames_o, mask_o,
               dur_v, csum_v, idxg_v, a_v, b_v, c_v, msk_v, mf_v,
               sbufA, mbufA, ebufA, sbufB, mbufB, ebufB, zbuf,
               isem, osem, zsem, msem):
    cid = lax.axis_index("c")
    sid = lax.axis_index("s")
    wid = cid * 16 + sid
    b = wid // TPB
    sub = wid % TPB

    pltpu.sync_copy(dur.at[b], dur_v)
    pltpu.sync_copy(mf, mf_v)

    def cs_step(i, carry):
        v = dur_v[pl.ds(i * 16, 16)]
        csum_v[pl.ds(i * 16, 16)] = jnp.cumsum(v) + carry
        return carry + jnp.sum(v)

    total = lax.fori_loop(0, N // 16, cs_step, jnp.int32(0))
    vt_vec = jnp.minimum(jnp.full((16,), total, jnp.int32), mf_v[...])
    vt_s = jnp.max(vt_vec)

    def z_step(i, _):
        for k in range(F // 16):
            zbuf[i, pl.ds(k * 16, 16)] = jnp.zeros((16,), jnp.float32)
        return 0

    lax.fori_loop(0, CH, z_step, 0)

    def pre_step(i, _):
        l = i * 16
        j = l // CH
        w = l - j * CH
        g = sub + TPB * j
        tvec = g * CH + w + lax.iota(jnp.int32, 16)
        lo = jnp.zeros((16,), jnp.int32)
        hi = jnp.full((16,), N, jnp.int32)
        for _step in range(12):
            m = jnp.minimum((lo + hi) >> 1, N - 1)
            vals = plsc.load_gather(csum_v, [m])
            pred = vals <= tvec
            lo = jnp.where(pred, m + 1, lo)
            hi = jnp.where(pred, hi, m)
        idx_c = jnp.minimum(lo, N - 1)
        d = plsc.load_gather(dur_v, [idx_c])
        cs = plsc.load_gather(csum_v, [idx_c])
        p = tvec - (cs - d)
        half = d >> 1
        rem = d - half
        pf = p.astype(jnp.float32)
        halff = half.astype(jnp.float32)
        den1 = jnp.maximum(half - 1, 1).astype(jnp.float32)
        den2 = jnp.maximum(rem - 1, 1).astype(jnp.float32)
        t1 = jnp.where(half > 1, pf / den1, 0.0)
        t2 = jnp.where(rem > 1, (pf - halff) / den2, 0.0)
        s1 = p < half
        a3 = jnp.where(s1, 1.0 - t1, 0.0)
        b3 = jnp.where(s1, t1, 1.0 - t2)
        c3 = jnp.where(s1, 0.0, t2)
        a2 = jnp.where(p == 0, 1.0, 0.0)
        is1 = d == 1
        is2 = d == 2
        a = jnp.where(is1, 0.0, jnp.where(is2, a2, a3))
        bb = jnp.where(is1, 1.0, jnp.where(is2, 0.0, b3))
        c = jnp.where(is1, 0.0, jnp.where(is2, 1.0 - a2, c3))
        valid = tvec < vt_vec
        vf = jnp.where(valid, 1.0, 0.0)
        sl = pl.ds(l, 16)
        a_v[sl] = a * vf
        b_v[sl] = bb * vf
        c_v[sl] = c * vf
        msk_v[sl] = jnp.where(valid, 1, 0)
        idxg_v[sl] = b * N + idx_c
        return 0

    lax.fori_loop(0, FPT // 16, pre_step, 0)

    num = jnp.maximum(vt_s - sub * CH, 0)
    nv = jnp.minimum(jnp.int32(NCH), (num + CH * TPB - 1) // (CH * TPB))

    def row0_of(j):
        return b * T + (sub + TPB * j) * CH

    def pre_out(j, _):
        loc = j * CH
        r0 = row0_of(j)
        pltpu.async_copy(msk_v.at[pl.ds(loc, CH)],
                         mask_o.at[pl.ds(r0, CH)], msem)

        @pl.when(j >= nv)
        def _zero_chunk():
            pltpu.async_copy(zbuf, frames_o.at[pl.ds(r0, CH)], zsem)

        return 0

    lax.fori_loop(0, NCH, pre_out, 0)

    bufs = ((sbufA, mbufA, ebufA), (sbufB, mbufB, ebufB))

    def fire_in(j, trio):
        idx_sl = idxg_v.at[pl.ds(j * CH, CH)]
        pltpu.async_copy(st.at[idx_sl], trio[0], isem)
        pltpu.async_copy(mi.at[idx_sl], trio[1], isem)
        pltpu.async_copy(en.at[idx_sl], trio[2], isem)

    def drain_in(trio):
        pltpu.make_async_copy(st.at[pl.ds(0, CH)], trio[0], isem).wait()
        pltpu.make_async_copy(mi.at[pl.ds(0, CH)], trio[1], isem).wait()
        pltpu.make_async_copy(en.at[pl.ds(0, CH)], trio[2], isem).wait()

    def drain_out_one():
        pltpu.make_async_copy(frames_o.at[pl.ds(0, CH)], sbufA, osem).wait()

    def process(i, trio):
        sb, mb, eb = trio
        loc = i * CH

        def f_step(fr, _c):
            iv = jnp.full((16,), loc + fr, jnp.int32)
            asp = plsc.load_gather(a_v, [iv])
            bsp = plsc.load_gather(b_v, [iv])
            csp = plsc.load_gather(c_v, [iv])
            for k in range(F // 16):
                fsl = pl.ds(k * 16, 16)
                sb[fr, fsl] = (sb[fr, fsl] * asp + mb[fr, fsl] * bsp
                               + eb[fr, fsl] * csp)
            return 0

        lax.fori_loop(0, CH, f_step, 0)
        pltpu.async_copy(sb, frames_o.at[pl.ds(row0_of(i), CH)], osem)

    @pl.when(nv > 0)
    def _prologue():
        fire_in(0, bufs[0])

    def ring_step(i, _):
        @pl.when(i >= 1)
        def _free_other():
            drain_out_one()

        for par in (0, 1):
            @pl.when(i % 2 == par)
            def _sub(par=par):
                @pl.when(i + 1 < nv)
                def _prefetch():
                    fire_in(i + 1, bufs[1 - par])

                drain_in(bufs[par])
                process(i, bufs[par])

        return 0

    lax.fori_loop(0, nv, ring_step, 0)

    @pl.when(nv > 0)
    def _last_out():
        drain_out_one()

    def drain_z(i, _):
        pltpu.make_async_copy(frames_o.at[pl.ds(0, CH)], zbuf, zsem).wait()
        return 0

    lax.fori_loop(0, jnp.int32(NCH) - nv, drain_z, 0)

    def drain_m(i, _):
        pltpu.make_async_copy(mask_o.at[pl.ds(0, CH)],
                              msk_v.at[pl.ds(0, CH)], msem).wait()
        return 0

    lax.fori_loop(0, NCH, drain_m, 0)


def kernel(start, mid, end, durations, max_frames):
    st = start.reshape(B * N, F)
    mi = mid.reshape(B * N, F)
    en = end.reshape(B * N, F)
    dur = durations.astype(jnp.int32)
    mf = jnp.full((16,), jnp.asarray(max_frames, jnp.int32))
    frames_flat, mask_i = _sc_interp(st, mi, en, dur, mf)
    frames = frames_flat.reshape(B, T, F)
    mask = mask_i.reshape(B, T) != 0
    return frames, mask


# speedup vs baseline: 141.6781x; 1.1266x over previous
import functools

import jax
import jax.numpy as jnp
from jax import lax
from jax.experimental import pallas as pl
from jax.experimental.pallas import tpu as pltpu
from jax.experimental.pallas import tpu_sc as plsc

B, N, F = 8, 2048, 256
T = 8192
NW = 32
TPB = NW // B
CH = 64
NCH = T // (TPB * CH)
FPT = T // TPB

_mesh = plsc.VectorSubcoreMesh(core_axis_name="c", subcore_axis_name="s")


@functools.partial(
    pl.kernel,
    out_type=[
        jax.ShapeDtypeStruct((B * T, F), jnp.float32),
        jax.ShapeDtypeStruct((B * T,), jnp.int32),
    ],
    mesh=_mesh,
    compiler_params=pltpu.CompilerParams(needs_layout_passes=False),
    scratch_types=[
        pltpu.VMEM((N,), jnp.int32),
        pltpu.VMEM((N,), jnp.int32),
        pltpu.VMEM((FPT,), jnp.int32),
        pltpu.VMEM((FPT,), jnp.float32),
        pltpu.VMEM((FPT,), jnp.float32),
        pltpu.VMEM((FPT,), jnp.float32),
        pltpu.VMEM((FPT,), jnp.int32),
        pltpu.VMEM((16,), jnp.int32),
        pltpu.VMEM((CH, F), jnp.float32),
        pltpu.VMEM((CH, F), jnp.float32),
        pltpu.VMEM((CH, F), jnp.float32),
        pltpu.VMEM((CH, F), jnp.float32),
        pltpu.VMEM((CH, F), jnp.float32),
        pltpu.VMEM((CH, F), jnp.float32),
        pltpu.VMEM((CH, F), jnp.float32),
        pltpu.VMEM((CH,), jnp.int32),
        pltpu.SemaphoreType.DMA,
        pltpu.SemaphoreType.DMA,
        pltpu.SemaphoreType.DMA,
        pltpu.SemaphoreType.DMA,
    ],
)
def _sc_interp(st, mi, en, dur, mf, frames_o, mask_o,
               dur_v, csum_v, idxg_v, a_v, b_v, c_v, msk_v, mf_v,
               sbufA, mbufA, ebufA, sbufB, mbufB, ebufB, zbuf, zmask_v,
               isem, osem, zsem, msem):
    cid = lax.axis_index("c")
    sid = lax.axis_index("s")
    wid = cid * 16 + sid
    b = wid // TPB
    sub = wid % TPB

    pltpu.sync_copy(dur.at[b], dur_v)
    pltpu.sync_copy(mf, mf_v)

    def cs_step(i, carry):
        v = dur_v[pl.ds(i * 16, 16)]
        csum_v[pl.ds(i * 16, 16)] = jnp.cumsum(v) + carry
        return carry + jnp.sum(v)

    total = lax.fori_loop(0, N // 16, cs_step, jnp.int32(0))
    vt_vec = jnp.minimum(jnp.full((16,), total, jnp.int32), mf_v[...])
    vt_s = jnp.max(vt_vec)

    def z_step(i, _):
        for k in range(F // 16):
            zbuf[i, pl.ds(k * 16, 16)] = jnp.zeros((16,), jnp.float32)
        return 0

    lax.fori_loop(0, CH, z_step, 0)
    for k in range(CH // 16):
        zmask_v[pl.ds(k * 16, 16)] = jnp.zeros((16,), jnp.int32)

    def pre_step(i, _):
        l = i * 16
        j = l // CH
        w = l - j * CH
        g = sub + TPB * j
        tvec = g * CH + w + lax.iota(jnp.int32, 16)
        lo = jnp.zeros((16,), jnp.int32)
        hi = jnp.full((16,), N, jnp.int32)
        for _step in range(12):
            m = jnp.minimum((lo + hi) >> 1, N - 1)
            vals = plsc.load_gather(csum_v, [m])
            pred = vals <= tvec
            lo = jnp.where(pred, m + 1, lo)
            hi = jnp.where(pred, hi, m)
        idx_c = jnp.minimum(lo, N - 1)
        d = plsc.load_gather(dur_v, [idx_c])
        cs = plsc.load_gather(csum_v, [idx_c])
        p = tvec - (cs - d)
        half = d >> 1
        rem = d - half
        pf = p.astype(jnp.float32)
        halff = half.astype(jnp.float32)
        den1 = jnp.maximum(half - 1, 1).astype(jnp.float32)
        den2 = jnp.maximum(rem - 1, 1).astype(jnp.float32)
        t1 = jnp.where(half > 1, pf / den1, 0.0)
        t2 = jnp.where(rem > 1, (pf - halff) / den2, 0.0)
        s1 = p < half
        a3 = jnp.where(s1, 1.0 - t1, 0.0)
        b3 = jnp.where(s1, t1, 1.0 - t2)
        c3 = jnp.where(s1, 0.0, t2)
        a2 = jnp.where(p == 0, 1.0, 0.0)
        is1 = d == 1
        is2 = d == 2
        a = jnp.where(is1, 0.0, jnp.where(is2, a2, a3))
        bb = jnp.where(is1, 1.0, jnp.where(is2, 0.0, b3))
        c = jnp.where(is1, 0.0, jnp.where(is2, 1.0 - a2, c3))
        valid = tvec < vt_vec
        vf = jnp.where(valid, 1.0, 0.0)
        sl = pl.ds(l, 16)
        a_v[sl] = a * vf
        b_v[sl] = bb * vf
        c_v[sl] = c * vf
        msk_v[sl] = jnp.where(valid, 1, 0)
        idxg_v[sl] = b * N + idx_c
        return 0

    def precompute_chunk(c):
        lax.fori_loop(c * (CH // 16), (c + 1) * (CH // 16), pre_step, 0)

    num = jnp.maximum(vt_s - sub * CH, 0)
    nv = jnp.minimum(jnp.int32(NCH), (num + CH * TPB - 1) // (CH * TPB))

    def row0_of(j):
        return b * T + (sub + TPB * j) * CH

    def fire_mask(j):
        pltpu.async_copy(msk_v.at[pl.ds(j * CH, CH)],
                         mask_o.at[pl.ds(row0_of(j), CH)], msem)

    def pre_out(j, _):
        r0 = row0_of(j)

        @pl.when(j >= nv)
        def _zero_chunk():
            pltpu.async_copy(zbuf, frames_o.at[pl.ds(r0, CH)], zsem)
            pltpu.async_copy(zmask_v, mask_o.at[pl.ds(r0, CH)], msem)

        return 0

    lax.fori_loop(0, NCH, pre_out, 0)

    bufs = ((sbufA, mbufA, ebufA), (sbufB, mbufB, ebufB))

    def fire_in(j, trio):
        idx_sl = idxg_v.at[pl.ds(j * CH, CH)]
        pltpu.async_copy(st.at[idx_sl], trio[0], isem)
        pltpu.async_copy(mi.at[idx_sl], trio[1], isem)
        pltpu.async_copy(en.at[idx_sl], trio[2], isem)

    def drain_in(trio):
        pltpu.make_async_copy(st.at[pl.ds(0, CH)], trio[0], isem).wait()
        pltpu.make_async_copy(mi.at[pl.ds(0, CH)], trio[1], isem).wait()
        pltpu.make_async_copy(en.at[pl.ds(0, CH)], trio[2], isem).wait()

    def drain_out_one():
        pltpu.make_async_copy(frames_o.at[pl.ds(0, CH)], sbufA, osem).wait()

    def process(i, trio):
        sb, mb, eb = trio
        loc = i * CH

        def f_step(fr, _c):
            iv = jnp.full((16,), loc + fr, jnp.int32)
            asp = plsc.load_gather(a_v, [iv])
            bsp = plsc.load_gather(b_v, [iv])
            csp = plsc.load_gather(c_v, [iv])
            for k in range(F // 16):
                fsl = pl.ds(k * 16, 16)
                sb[fr, fsl] = (sb[fr, fsl] * asp + mb[fr, fsl] * bsp
                               + eb[fr, fsl] * csp)
            return 0

        lax.fori_loop(0, CH, f_step, 0)
        pltpu.async_copy(sb, frames_o.at[pl.ds(row0_of(i), CH)], osem)

    @pl.when(nv > 0)
    def _prologue():
        precompute_chunk(jnp.int32(0))
        fire_mask(jnp.int32(0))
        fire_in(0, bufs[0])

    def ring_step(i, _):
        @pl.when(i >= 1)
        def _free_other():
            drain_out_one()

        @pl.when(i + 1 < nv)
        def _pre_next():
            precompute_chunk(i + 1)
            fire_mask(i + 1)

        for par in (0, 1):
            @pl.when(i % 2 == par)
            def _sub(par=par):
                @pl.when(i + 1 < nv)
                def _prefetch():
                    fire_in(i + 1, bufs[1 - par])

                drain_in(bufs[par])
                process(i, bufs[par])

        return 0

    lax.fori_loop(0, nv, ring_step, 0)

    @pl.when(nv > 0)
    def _last_out():
        drain_out_one()

    def drain_z(i, _):
        pltpu.make_async_copy(frames_o.at[pl.ds(0, CH)], zbuf, zsem).wait()
        return 0

    lax.fori_loop(0, jnp.int32(NCH) - nv, drain_z, 0)

    def drain_m(i, _):
        pltpu.make_async_copy(mask_o.at[pl.ds(0, CH)],
                              msk_v.at[pl.ds(0, CH)], msem).wait()
        return 0

    lax.fori_loop(0, NCH, drain_m, 0)


def kernel(start, mid, end, durations, max_frames):
    st = start.reshape(B * N, F)
    mi = mid.reshape(B * N, F)
    en = end.reshape(B * N, F)
    dur = durations.astype(jnp.int32)
    mf = jnp.full((16,), jnp.asarray(max_frames, jnp.int32))
    frames_flat, mask_i = _sc_interp(st, mi, en, dur, mf)
    frames = frames_flat.reshape(B, T, F)
    mask = mask_i.reshape(B, T) != 0
    return frames, mask
